# baseline (speedup 1.0000x reference)
import functools

import jax
import jax.numpy as jnp
from jax import lax
from jax.experimental import pallas as pl
from jax.experimental.pallas import tpu as pltpu
from jax.experimental.pallas import tpu_sc as plsc

_NUM_NODES = 1000000
_DIM = 32
_HALF = _DIM // 2
_BATCH = 16384

_NC = 2
_NS = 16
_NW = _NC * _NS
_PAIRS_PER_W = _BATCH // _NW
_ROWS_PER_W = 2 * _PAIRS_PER_W
_CHUNK = 128
_NCHUNK = _ROWS_PER_W // _CHUNK
_BLOCKS = _PAIRS_PER_W // 16


def _body(ids_hbm, tl_hbm, tr_hbm, out_hbm, idx_v, rowsl_v, rowsr_v, out_v,
          sem):
    wid = lax.axis_index("s") * _NC + lax.axis_index("c")

    pltpu.sync_copy(ids_hbm.at[pl.ds(wid * _NCHUNK, _NCHUNK), :], idx_v)

    copies = []
    for j in range(_NCHUNK):
        sl = pl.ds(j * _CHUNK, _CHUNK)
        copies.append(
            pltpu.async_copy(tl_hbm.at[idx_v.at[j]], rowsl_v.at[sl, :], sem)
        )
        copies.append(
            pltpu.async_copy(tr_hbm.at[idx_v.at[j]], rowsr_v.at[sl, :], sem)
        )
    for c in copies:
        c.wait()

    lanes = lax.broadcasted_iota(jnp.int32, (16,), 0)

    def block(b, _):
        row_a = 32 * b + 2 * lanes
        row_b = row_a + 1
        acc = jnp.zeros((16,), jnp.float32)
        for rows in (rowsl_v, rowsr_v):
            for j in range(_HALF):
                col = jnp.full((16,), j, jnp.int32)
                a = plsc.load_gather(rows, [row_a, col])
                bb = plsc.load_gather(rows, [row_b, col])
                d = a - bb
                acc = acc + d * d
        out_v[pl.ds(b * 16, 16)] = acc
        return _

    lax.fori_loop(0, _BLOCKS, block, None)

    pltpu.sync_copy(out_v, out_hbm.at[pl.ds(wid * _PAIRS_PER_W, _PAIRS_PER_W)])


@jax.jit
def kernel(inputs, embedding_table):
    ids2d = inputs.astype(jnp.int32).reshape(_NW * _NCHUNK, _CHUNK)
    tl = embedding_table[:, :_HALF]
    tr = embedding_table[:, _HALF:]
    run = functools.partial(
        pl.kernel,
        mesh=plsc.VectorSubcoreMesh(core_axis_name="c", subcore_axis_name="s"),
        out_type=jax.ShapeDtypeStruct((_BATCH,), jnp.float32),
        compiler_params=pltpu.CompilerParams(
            needs_layout_passes=False, use_tc_tiling_on_sc=False
        ),
        scratch_types=[
            pltpu.VMEM((_NCHUNK, _CHUNK), jnp.int32),
            pltpu.VMEM((_ROWS_PER_W, _HALF), jnp.float32),
            pltpu.VMEM((_ROWS_PER_W, _HALF), jnp.float32),
            pltpu.VMEM((_PAIRS_PER_W,), jnp.float32),
            pltpu.SemaphoreType.DMA,
        ],
    )(_body)
    return run(ids2d, tl, tr)


# speedup vs baseline: 2.3609x; 2.3609x over previous
import functools

import jax
import jax.numpy as jnp
from jax import lax
from jax.experimental import pallas as pl
from jax.experimental.pallas import tpu as pltpu
from jax.experimental.pallas import tpu_sc as plsc

_NUM_NODES = 1000000
_DIM = 32
_BATCH = 16384

_NC = 2
_NS = 16
_NW = _NC * _NS
_PAIRS_PER_W = _BATCH // _NW
_ROWS_PER_W = 2 * _PAIRS_PER_W
_CHUNK = 128
_NCHUNK = _ROWS_PER_W // _CHUNK
_BLOCKS = _PAIRS_PER_W // 16


def _body(ids_hbm, table_hbm, out_hbm, idx_v, rows_v, out_v, sem):
    wid = lax.axis_index("s") * _NC + lax.axis_index("c")

    pltpu.sync_copy(ids_hbm.at[pl.ds(wid * _NCHUNK, _NCHUNK), :], idx_v)

    copies = []
    for j in range(_NCHUNK):
        copies.append(
            pltpu.async_copy(
                table_hbm.at[idx_v.at[j]],
                rows_v.at[pl.ds(j * _CHUNK, _CHUNK), :],
                sem,
            )
        )
    for c in copies:
        c.wait()

    lanes = lax.broadcasted_iota(jnp.int32, (16,), 0)

    def block(b, _):
        row_a = 32 * b + 2 * lanes
        row_b = row_a + 1
        acc = jnp.zeros((16,), jnp.float32)
        for j in range(_DIM):
            col = jnp.full((16,), j, jnp.int32)
            a = plsc.load_gather(rows_v, [row_a, col])
            bb = plsc.load_gather(rows_v, [row_b, col])
            d = a - bb
            acc = acc + d * d
        out_v[pl.ds(b * 16, 16)] = acc
        return _

    lax.fori_loop(0, _BLOCKS, block, None)

    pltpu.sync_copy(out_v, out_hbm.at[pl.ds(wid * _PAIRS_PER_W, _PAIRS_PER_W)])


@jax.jit
def kernel(inputs, embedding_table):
    ids2d = inputs.astype(jnp.int32).reshape(_NW * _NCHUNK, _CHUNK)
    run = functools.partial(
        pl.kernel,
        mesh=plsc.VectorSubcoreMesh(core_axis_name="c", subcore_axis_name="s"),
        out_type=jax.ShapeDtypeStruct((_BATCH,), jnp.float32),
        compiler_params=pltpu.CompilerParams(
            needs_layout_passes=False, use_tc_tiling_on_sc=False
        ),
        scratch_types=[
            pltpu.VMEM((_NCHUNK, _CHUNK), jnp.int32),
            pltpu.VMEM((_ROWS_PER_W, _DIM), jnp.float32),
            pltpu.VMEM((_PAIRS_PER_W,), jnp.float32),
            pltpu.SemaphoreType.DMA,
        ],
    )(_body)
    return run(ids2d, embedding_table)


# speedup vs baseline: 2.3688x; 1.0034x over previous
import functools

import jax
import jax.numpy as jnp
from jax import lax
from jax.experimental import pallas as pl
from jax.experimental.pallas import tpu as pltpu
from jax.experimental.pallas import tpu_sc as plsc

_NUM_NODES = 1000000
_DIM = 32
_BATCH = 16384

_NC = 2
_NS = 16
_NW = _NC * _NS
_PAIRS_PER_W = _BATCH // _NW
_ROWS_PER_W = 2 * _PAIRS_PER_W
_CHUNK = 128
_NCHUNK = _ROWS_PER_W // _CHUNK
_BLOCKS = _PAIRS_PER_W // 16


def _body(ids_hbm, table_hbm, out_hbm, idx_v, rows_v, out_v, sem):
    wid = lax.axis_index("s") * _NC + lax.axis_index("c")

    pltpu.sync_copy(ids_hbm.at[pl.ds(wid * _NCHUNK, _NCHUNK), :], idx_v)

    copies = []
    for j in range(_NCHUNK):
        copies.append(
            pltpu.async_copy(
                table_hbm.at[idx_v.at[j]],
                rows_v.at[pl.ds(j * _CHUNK, _CHUNK), :],
                sem,
            )
        )
    for c in copies:
        c.wait()

    lanes = lax.broadcasted_iota(jnp.int32, (16,), 0)

    def block(b, _):
        row_a = 32 * b + 2 * lanes
        row_b = row_a + 1
        acc = jnp.zeros((16,), jnp.float32)
        for j in range(_DIM):
            col = jnp.full((16,), j, jnp.int32)
            a = plsc.load_gather(rows_v, [row_a, col])
            bb = plsc.load_gather(rows_v, [row_b, col])
            d = a - bb
            acc = acc + d * d
        out_v[pl.ds(b * 16, 16)] = acc
        return _

    lax.fori_loop(0, _BLOCKS, block, None)

    pltpu.sync_copy(out_v, out_hbm.at[pl.ds(wid * _PAIRS_PER_W, _PAIRS_PER_W)])


@jax.jit
def kernel(inputs, embedding_table):
    ids2d = inputs.astype(jnp.int32).reshape(_NW * _NCHUNK, _CHUNK)
    table = embedding_table * jnp.float32(1.0)
    run = functools.partial(
        pl.kernel,
        mesh=plsc.VectorSubcoreMesh(core_axis_name="c", subcore_axis_name="s"),
        out_type=jax.ShapeDtypeStruct((_BATCH,), jnp.float32),
        compiler_params=pltpu.CompilerParams(
            needs_layout_passes=False, use_tc_tiling_on_sc=False
        ),
        scratch_types=[
            pltpu.VMEM((_NCHUNK, _CHUNK), jnp.int32),
            pltpu.VMEM((_ROWS_PER_W, _DIM), jnp.float32),
            pltpu.VMEM((_PAIRS_PER_W,), jnp.float32),
            pltpu.SemaphoreType.DMA,
        ],
    )(_body)
    return run(ids2d, table)
